# baseline (speedup 1.0000x reference)
import jax
import jax.numpy as jnp
from jax.experimental import pallas as pl
from jax.experimental.pallas import tpu as pltpu

_BM = 2048
_NBUF = 8


def _copy(x_hbm, xbuf, sems, block, slot):
    return pltpu.make_async_copy(
        x_hbm.at[pl.ds(block * _BM, _BM), :], xbuf.at[slot], sems.at[slot])


def _gate_matmul(x_hbm, wg_ref, o_ref, xbuf, wgt, sems):
    i = pl.program_id(0)
    nsteps = pl.num_programs(0)

    @pl.when(i == 0)
    def _warmup():
        for b in range(_NBUF):
            _copy(x_hbm, xbuf, sems, b, b).start()
        wgt[...] = wg_ref[...].T

    slot = jax.lax.rem(i, _NBUF)
    _copy(x_hbm, xbuf, sems, i, slot).wait()
    o_ref[...] = jnp.dot(xbuf[slot], wgt[...],
                         preferred_element_type=jnp.float32).T

    nxt = i + _NBUF

    @pl.when(nxt < nsteps)
    def _prefetch():
        _copy(x_hbm, xbuf, sems, nxt, slot).start()


def kernel(x, wg):
    m, k = x.shape
    e = wg.shape[0]
    out_t = pl.pallas_call(
        _gate_matmul,
        grid=(m // _BM,),
        in_specs=[
            pl.BlockSpec(memory_space=pl.ANY),
            pl.BlockSpec((e, k), lambda i: (0, 0)),
        ],
        out_specs=pl.BlockSpec((e, _BM), lambda i: (0, i)),
        out_shape=jax.ShapeDtypeStruct((e, m), jnp.float32),
        scratch_shapes=[
            pltpu.VMEM((_NBUF, _BM, k), jnp.float32),
            pltpu.VMEM((k, e), jnp.float32),
            pltpu.SemaphoreType.DMA((_NBUF,)),
        ],
    )(x, wg)
    return out_t.T


# speedup vs baseline: 1.0082x; 1.0082x over previous
import jax
import jax.numpy as jnp
from jax.experimental import pallas as pl
from jax.experimental.pallas import tpu as pltpu

_BM = 1024
_NBUF = 8


def _copy(x_hbm, xbuf, sems, block, slot):
    return pltpu.make_async_copy(
        x_hbm.at[pl.ds(block * _BM, _BM), :], xbuf.at[slot], sems.at[slot])


def _gate_matmul(x_hbm, wg_ref, o_ref, xbuf, wgt, sems):
    i = pl.program_id(0)
    nsteps = pl.num_programs(0)

    @pl.when(i == 0)
    def _warmup():
        for b in range(_NBUF):
            _copy(x_hbm, xbuf, sems, b, b).start()
        wgt[...] = wg_ref[...].T.astype(jnp.bfloat16)

    slot = jax.lax.rem(i, _NBUF)
    _copy(x_hbm, xbuf, sems, i, slot).wait()
    o_ref[...] = jnp.dot(xbuf[slot].astype(jnp.bfloat16), wgt[...],
                         preferred_element_type=jnp.float32).T

    nxt = i + _NBUF

    @pl.when(nxt < nsteps)
    def _prefetch():
        _copy(x_hbm, xbuf, sems, nxt, slot).start()


def kernel(x, wg):
    m, k = x.shape
    e = wg.shape[0]
    out_t = pl.pallas_call(
        _gate_matmul,
        grid=(m // _BM,),
        in_specs=[
            pl.BlockSpec(memory_space=pl.ANY),
            pl.BlockSpec((e, k), lambda i: (0, 0)),
        ],
        out_specs=pl.BlockSpec((e, _BM), lambda i: (0, i)),
        out_shape=jax.ShapeDtypeStruct((e, m), jnp.float32),
        scratch_shapes=[
            pltpu.VMEM((_NBUF, _BM, k), jnp.float32),
            pltpu.VMEM((k, e), jnp.bfloat16),
            pltpu.SemaphoreType.DMA((_NBUF,)),
        ],
    )(x, wg)
    return out_t.T
